# Initial kernel scaffold; baseline (speedup 1.0000x reference)
import functools

import jax
import jax.numpy as jnp
from jax import lax
from jax.experimental import pallas as pl
from jax.experimental.pallas import tpu as pltpu
from jax.experimental.pallas import tpu_sc as plsc

N_NODES = 10000
N_EDGES = 320000
D_FEAT = 128
D_HIDDEN = 128
N_CLASSES = 64
ALPHA = 0.25

NC = 2
NS = 16
NW = NC * NS
EPW = N_EDGES // NW
CHUNK = 80
NITER = EPW // CHUNK
RPW = N_NODES // NS
DEG_W = 16

_mesh = plsc.VectorSubcoreMesh(core_axis_name="c", subcore_axis_name="s")


@functools.partial(
    pl.kernel,
    mesh=_mesh,
    out_type=jax.ShapeDtypeStruct((NC, N_NODES, DEG_W), jnp.float32),
    scratch_types=[
        pltpu.VMEM((CHUNK,), jnp.int32),
        pltpu.VMEM((CHUNK, DEG_W), jnp.float32),
        pltpu.VMEM_SHARED((N_NODES, DEG_W), jnp.float32),
    ],
)
def _sc_degree(rows_hbm, zeros_hbm, ones_hbm, out_hbm, idx_v, ones_v, acc_sh):
    cid = lax.axis_index("c")
    sid = lax.axis_index("s")
    wid = cid * NS + sid
    pltpu.sync_copy(zeros_hbm.at[pl.ds(sid * RPW, RPW)],
                    acc_sh.at[pl.ds(sid * RPW, RPW)])
    pltpu.sync_copy(ones_hbm, ones_v)
    plsc.subcore_barrier()

    @pl.loop(0, NITER)
    def _(i):
        base = wid * EPW + i * CHUNK
        pltpu.sync_copy(rows_hbm.at[pl.ds(base, CHUNK)], idx_v)
        pltpu.sync_copy(ones_v, acc_sh.at[idx_v], add=True)

    plsc.subcore_barrier()
    pltpu.sync_copy(acc_sh.at[pl.ds(sid * RPW, RPW)],
                    out_hbm.at[cid, pl.ds(sid * RPW, RPW)])


@functools.partial(
    pl.kernel,
    mesh=_mesh,
    out_type=jax.ShapeDtypeStruct((NC, N_NODES, N_CLASSES), jnp.float32),
    scratch_types=[
        pltpu.VMEM((CHUNK,), jnp.int32),
        pltpu.VMEM((CHUNK,), jnp.int32),
        pltpu.VMEM((CHUNK, N_CLASSES), jnp.float32),
        pltpu.VMEM_SHARED((N_NODES, N_CLASSES), jnp.float32),
        pltpu.SemaphoreType.DMA,
    ],
)
def _sc_propagate(logits_hbm, rows_hbm, cols_hbm, zeros_hbm, out_hbm,
                  cols_v, rows_v, gath_v, acc_sh, sem):
    cid = lax.axis_index("c")
    sid = lax.axis_index("s")
    wid = cid * NS + sid
    pltpu.sync_copy(zeros_hbm.at[pl.ds(sid * RPW, RPW)],
                    acc_sh.at[pl.ds(sid * RPW, RPW)])
    plsc.subcore_barrier()

    @pl.loop(0, NITER)
    def _(i):
        base = wid * EPW + i * CHUNK
        pltpu.sync_copy(cols_hbm.at[pl.ds(base, CHUNK)], cols_v)
        pltpu.sync_copy(rows_hbm.at[pl.ds(base, CHUNK)], rows_v)
        pltpu.async_copy(logits_hbm.at[cols_v], gath_v, sem).wait()
        pltpu.sync_copy(gath_v, acc_sh.at[rows_v], add=True)

    plsc.subcore_barrier()
    pltpu.sync_copy(acc_sh.at[pl.ds(sid * RPW, RPW)],
                    out_hbm.at[cid, pl.ds(sid * RPW, RPW)])


_BLK = 1000
_GRID = N_NODES // _BLK


def _mlp_body(x_ref, w1_ref, w2_ref, o_ref):
    h = jnp.dot(x_ref[...], w1_ref[...],
                preferred_element_type=jnp.float32,
                precision=lax.Precision.HIGHEST)
    h = jnp.maximum(h, 0.0)
    o_ref[...] = jnp.dot(h, w2_ref[...],
                         preferred_element_type=jnp.float32,
                         precision=lax.Precision.HIGHEST)


def _tc_mlp(x, W1, W2):
    return pl.pallas_call(
        _mlp_body,
        grid=(_GRID,),
        in_specs=[
            pl.BlockSpec((_BLK, D_FEAT), lambda i: (i, 0)),
            pl.BlockSpec((D_FEAT, D_HIDDEN), lambda i: (0, 0)),
            pl.BlockSpec((D_HIDDEN, N_CLASSES), lambda i: (0, 0)),
        ],
        out_specs=pl.BlockSpec((_BLK, N_CLASSES), lambda i: (i, 0)),
        out_shape=jax.ShapeDtypeStruct((N_NODES, N_CLASSES), jnp.float32),
    )(x, W1, W2)


def _combine_body(last, mp_ref, dp_ref, ll_ref, o_ref):
    msg = mp_ref[0] + mp_ref[1]
    deg = dp_ref[0, :, 0:1] + dp_ref[1, :, 0:1]
    coef = (1.0 - ALPHA) / jnp.maximum(deg, 1e-12)
    logits = coef * msg + ALPHA * ll_ref[...]
    if last:
        mx = jnp.max(logits, axis=1, keepdims=True)
        lse = jnp.log(jnp.sum(jnp.exp(logits - mx), axis=1, keepdims=True)) + mx
        o_ref[...] = logits - lse
    else:
        o_ref[...] = logits


def _tc_combine(msg_part, deg_part, local_logits, last):
    return pl.pallas_call(
        functools.partial(_combine_body, last),
        grid=(_GRID,),
        in_specs=[
            pl.BlockSpec((NC, _BLK, N_CLASSES), lambda i: (0, i, 0)),
            pl.BlockSpec((NC, _BLK, DEG_W), lambda i: (0, i, 0)),
            pl.BlockSpec((_BLK, N_CLASSES), lambda i: (i, 0)),
        ],
        out_specs=pl.BlockSpec((_BLK, N_CLASSES), lambda i: (i, 0)),
        out_shape=jax.ShapeDtypeStruct((N_NODES, N_CLASSES), jnp.float32),
    )(msg_part, deg_part, local_logits)


def kernel(x, edge_index, W1, W2):
    rows = edge_index[0]
    cols = edge_index[1]
    z64 = jnp.zeros((N_NODES, N_CLASSES), jnp.float32)
    z16 = jnp.zeros((N_NODES, DEG_W), jnp.float32)
    ones = jnp.ones((CHUNK, DEG_W), jnp.float32)

    local_logits = _tc_mlp(x, W1, W2)
    deg_part = _sc_degree(rows, z16, ones)
    msg1 = _sc_propagate(local_logits, rows, cols, z64)
    logits1 = _tc_combine(msg1, deg_part, local_logits, last=False)
    msg2 = _sc_propagate(logits1, rows, cols, z64)
    return _tc_combine(msg2, deg_part, local_logits, last=True)


# speedup vs baseline: 4.9891x; 4.9891x over previous
import functools

import jax
import jax.numpy as jnp
from jax import lax
from jax.experimental import pallas as pl
from jax.experimental.pallas import tpu as pltpu
from jax.experimental.pallas import tpu_sc as plsc

N_NODES = 10000
N_EDGES = 320000
D_FEAT = 128
D_HIDDEN = 128
N_CLASSES = 64
ALPHA = 0.25
PADW = 128

NC = 2
NS = 16
NW = NC * NS
EPW = N_EDGES // NW
CHUNK = 80
NITER = EPW // CHUNK
ZSLAB = 624
ZTAIL = N_NODES - NS * ZSLAB

_mesh = plsc.VectorSubcoreMesh(core_axis_name="c", subcore_axis_name="s")


def _rowwise_copy(src, dst, sid):
    base = pl.multiple_of(sid * ZSLAB, 8)
    pltpu.sync_copy(src.at[pl.ds(base, ZSLAB)], dst.at[pl.ds(base, ZSLAB)])

    @pl.when(sid == 0)
    def _():
        pltpu.sync_copy(src.at[pl.ds(NS * ZSLAB, ZTAIL)],
                        dst.at[pl.ds(NS * ZSLAB, ZTAIL)])


@functools.partial(
    pl.kernel,
    mesh=_mesh,
    out_type=jax.ShapeDtypeStruct((NC, N_NODES, PADW), jnp.float32),
    scratch_types=[
        pltpu.VMEM((CHUNK,), jnp.int32),
        pltpu.VMEM((CHUNK,), jnp.int32),
        pltpu.VMEM((CHUNK, PADW), jnp.float32),
        pltpu.VMEM_SHARED((N_NODES, PADW), jnp.float32),
        pltpu.SemaphoreType.DMA,
    ],
)
def _sc_propagate(logits_hbm, rows_hbm, cols_hbm, zeros_hbm, out_hbm,
                  cols_v, rows_v, gath_v, acc_sh, sem):
    cid = lax.axis_index("c")
    sid = lax.axis_index("s")
    wid = cid * NS + sid
    _rowwise_copy(zeros_hbm, acc_sh, sid)
    plsc.subcore_barrier()

    @pl.loop(0, NITER)
    def _(i):
        base = pl.multiple_of(wid * EPW + i * CHUNK, 8)
        pltpu.sync_copy(cols_hbm.at[pl.ds(base, CHUNK)], cols_v)
        pltpu.sync_copy(rows_hbm.at[pl.ds(base, CHUNK)], rows_v)
        pltpu.async_copy(logits_hbm.at[cols_v], gath_v, sem).wait()
        pltpu.sync_copy(gath_v, acc_sh.at[rows_v], add=True)

    plsc.subcore_barrier()
    _rowwise_copy(acc_sh, out_hbm.at[cid], sid)


_BLK = 1000
_GRID = N_NODES // _BLK


def _mlp_body(x_ref, w1_ref, w2_ref, o_ref):
    h = jnp.dot(x_ref[...], w1_ref[...],
                preferred_element_type=jnp.float32,
                precision=lax.Precision.HIGHEST)
    h = jnp.maximum(h, 0.0)
    logits = jnp.dot(h, w2_ref[...],
                     preferred_element_type=jnp.float32,
                     precision=lax.Precision.HIGHEST)
    lane = lax.broadcasted_iota(jnp.int32, (_BLK, N_CLASSES), 1)
    pad = jnp.where(lane == 0, 1.0, 0.0)
    o_ref[...] = jnp.concatenate([logits, pad], axis=1)


def _tc_mlp(x, W1, W2):
    return pl.pallas_call(
        _mlp_body,
        grid=(_GRID,),
        in_specs=[
            pl.BlockSpec((_BLK, D_FEAT), lambda i: (i, 0)),
            pl.BlockSpec((D_FEAT, D_HIDDEN), lambda i: (0, 0)),
            pl.BlockSpec((D_HIDDEN, N_CLASSES), lambda i: (0, 0)),
        ],
        out_specs=pl.BlockSpec((_BLK, PADW), lambda i: (i, 0)),
        out_shape=jax.ShapeDtypeStruct((N_NODES, PADW), jnp.float32),
    )(x, W1, W2)


def _combine_body(last, mp_ref, dp_ref, ll_ref, o_ref):
    msg = mp_ref[0, :, :N_CLASSES] + mp_ref[1, :, :N_CLASSES]
    deg = dp_ref[0, :, N_CLASSES:N_CLASSES + 1] + dp_ref[1, :, N_CLASSES:N_CLASSES + 1]
    coef = (1.0 - ALPHA) / jnp.maximum(deg, 1e-12)
    logits = coef * msg + ALPHA * ll_ref[:, :N_CLASSES]
    if last:
        mx = jnp.max(logits, axis=1, keepdims=True)
        lse = jnp.log(jnp.sum(jnp.exp(logits - mx), axis=1, keepdims=True)) + mx
        o_ref[...] = logits - lse
    else:
        lane = lax.broadcasted_iota(jnp.int32, (_BLK, N_CLASSES), 1)
        pad = jnp.where(lane == 0, 1.0, 0.0)
        o_ref[...] = jnp.concatenate([logits, pad], axis=1)


def _tc_combine(msg_part, deg_part, local_logits, last):
    ow = N_CLASSES if last else PADW
    return pl.pallas_call(
        functools.partial(_combine_body, last),
        grid=(_GRID,),
        in_specs=[
            pl.BlockSpec((NC, _BLK, PADW), lambda i: (0, i, 0)),
            pl.BlockSpec((NC, _BLK, PADW), lambda i: (0, i, 0)),
            pl.BlockSpec((_BLK, PADW), lambda i: (i, 0)),
        ],
        out_specs=pl.BlockSpec((_BLK, ow), lambda i: (i, 0)),
        out_shape=jax.ShapeDtypeStruct((N_NODES, ow), jnp.float32),
    )(msg_part, deg_part, local_logits)


def kernel(x, edge_index, W1, W2):
    rows = edge_index[0]
    cols = edge_index[1]
    zpad = jnp.zeros((N_NODES, PADW), jnp.float32)

    local_logits = _tc_mlp(x, W1, W2)
    msg1 = _sc_propagate(local_logits, rows, cols, zpad)
    logits1 = _tc_combine(msg1, msg1, local_logits, last=False)
    msg2 = _sc_propagate(logits1, rows, cols, zpad)
    return _tc_combine(msg2, msg1, local_logits, last=True)


# speedup vs baseline: 8.7409x; 1.7520x over previous
import functools

import jax
import jax.numpy as jnp
from jax import lax
from jax.experimental import pallas as pl
from jax.experimental.pallas import tpu as pltpu
from jax.experimental.pallas import tpu_sc as plsc

N_NODES = 10000
N_EDGES = 320000
D_FEAT = 128
D_HIDDEN = 128
N_CLASSES = 64
ALPHA = 0.25
PADW = 128

NC = 2
NS = 16
NW = NC * NS
EPW = N_EDGES // NW
CHUNK = 80
NITER = EPW // CHUNK
ZSLAB = 624
ZTAIL = N_NODES - NS * ZSLAB

_mesh = plsc.VectorSubcoreMesh(core_axis_name="c", subcore_axis_name="s")


def _rowwise_copy(src, dst, sid):
    base = pl.multiple_of(sid * ZSLAB, 8)
    pltpu.sync_copy(src.at[pl.ds(base, ZSLAB)], dst.at[pl.ds(base, ZSLAB)])

    @pl.when(sid == 0)
    def _():
        pltpu.sync_copy(src.at[pl.ds(NS * ZSLAB, ZTAIL)],
                        dst.at[pl.ds(NS * ZSLAB, ZTAIL)])


@functools.partial(
    pl.kernel,
    mesh=_mesh,
    out_type=jax.ShapeDtypeStruct((NC, N_NODES, PADW), jnp.float32),
    scratch_types=[
        pltpu.VMEM((CHUNK,), jnp.int32),
        pltpu.VMEM((CHUNK,), jnp.int32),
        pltpu.VMEM((CHUNK,), jnp.int32),
        pltpu.VMEM((CHUNK,), jnp.int32),
        pltpu.VMEM((CHUNK, PADW), jnp.float32),
        pltpu.VMEM((CHUNK, PADW), jnp.float32),
        pltpu.VMEM_SHARED((N_NODES, PADW), jnp.float32),
        pltpu.SemaphoreType.DMA,
        pltpu.SemaphoreType.DMA,
        pltpu.SemaphoreType.DMA,
        pltpu.SemaphoreType.DMA,
        pltpu.SemaphoreType.DMA,
        pltpu.SemaphoreType.DMA,
    ],
)
def _sc_propagate(logits_hbm, rows_hbm, cols_hbm, zeros_hbm, out_hbm,
                  c0, c1, r0, r1, g0, g1, acc_sh,
                  sc0, sc1, sr0, sr1, sg0, sg1):
    cid = lax.axis_index("c")
    sid = lax.axis_index("s")
    wid = cid * NS + sid
    cbuf = (c0, c1)
    rbuf = (r0, r1)
    gbuf = (g0, g1)
    semc = (sc0, sc1)
    semr = (sr0, sr1)
    semg = (sg0, sg1)

    def ebase(k):
        return pl.multiple_of(wid * EPW + k * CHUNK, 8)

    def cols_copy(k, b):
        return pltpu.make_async_copy(
            cols_hbm.at[pl.ds(ebase(k), CHUNK)], cbuf[b], semc[b])

    def rows_copy(k, b):
        return pltpu.make_async_copy(
            rows_hbm.at[pl.ds(ebase(k), CHUNK)], rbuf[b], semr[b])

    def gath_copy(b):
        return pltpu.make_async_copy(
            logits_hbm.at[cbuf[b]], gbuf[b], semg[b])

    def scat(b):
        pltpu.sync_copy(gbuf[b], acc_sh.at[rbuf[b]], add=True)

    for b in range(2):
        cols_copy(b, b).start()
        rows_copy(b, b).start()
    _rowwise_copy(zeros_hbm, acc_sh, sid)
    plsc.subcore_barrier()
    cols_copy(0, 0).wait()
    gath_copy(0).start()

    def body(k, b, nxt_gather, nxt_idx):
        gath_copy(b).wait()
        if nxt_gather:
            cols_copy(k + 1, 1 - b).wait()
            gath_copy(1 - b).start()
        rows_copy(k, b).wait()
        scat(b)
        if nxt_idx:
            cols_copy(k + 2, b).start()
            rows_copy(k + 2, b).start()

    @pl.loop(0, (NITER - 3) // 2)
    def _(gi):
        for b in range(2):
            body(2 * gi + b, b, True, True)

    body(NITER - 3, 0, True, True)
    body(NITER - 2, 1, True, False)
    body(NITER - 1, 0, False, False)

    plsc.subcore_barrier()
    _rowwise_copy(acc_sh, out_hbm.at[cid], sid)


_BLK = 1000
_GRID = N_NODES // _BLK


def _mlp_body(x_ref, w1_ref, w2_ref, o_ref):
    h = jnp.dot(x_ref[...], w1_ref[...],
                preferred_element_type=jnp.float32,
                precision=lax.Precision.HIGHEST)
    h = jnp.maximum(h, 0.0)
    logits = jnp.dot(h, w2_ref[...],
                     preferred_element_type=jnp.float32,
                     precision=lax.Precision.HIGHEST)
    lane = lax.broadcasted_iota(jnp.int32, (_BLK, N_CLASSES), 1)
    pad = jnp.where(lane == 0, 1.0, 0.0)
    o_ref[...] = jnp.concatenate([logits, pad], axis=1)


def _tc_mlp(x, W1, W2):
    return pl.pallas_call(
        _mlp_body,
        grid=(_GRID,),
        in_specs=[
            pl.BlockSpec((_BLK, D_FEAT), lambda i: (i, 0)),
            pl.BlockSpec((D_FEAT, D_HIDDEN), lambda i: (0, 0)),
            pl.BlockSpec((D_HIDDEN, N_CLASSES), lambda i: (0, 0)),
        ],
        out_specs=pl.BlockSpec((_BLK, PADW), lambda i: (i, 0)),
        out_shape=jax.ShapeDtypeStruct((N_NODES, PADW), jnp.float32),
    )(x, W1, W2)


def _combine_body(last, mp_ref, dp_ref, ll_ref, o_ref):
    msg = mp_ref[0, :, :N_CLASSES] + mp_ref[1, :, :N_CLASSES]
    deg = dp_ref[0, :, N_CLASSES:N_CLASSES + 1] + dp_ref[1, :, N_CLASSES:N_CLASSES + 1]
    coef = (1.0 - ALPHA) / jnp.maximum(deg, 1e-12)
    logits = coef * msg + ALPHA * ll_ref[:, :N_CLASSES]
    if last:
        mx = jnp.max(logits, axis=1, keepdims=True)
        lse = jnp.log(jnp.sum(jnp.exp(logits - mx), axis=1, keepdims=True)) + mx
        o_ref[...] = logits - lse
    else:
        lane = lax.broadcasted_iota(jnp.int32, (_BLK, N_CLASSES), 1)
        pad = jnp.where(lane == 0, 1.0, 0.0)
        o_ref[...] = jnp.concatenate([logits, pad], axis=1)


def _tc_combine(msg_part, deg_part, local_logits, last):
    ow = N_CLASSES if last else PADW
    return pl.pallas_call(
        functools.partial(_combine_body, last),
        grid=(_GRID,),
        in_specs=[
            pl.BlockSpec((NC, _BLK, PADW), lambda i: (0, i, 0)),
            pl.BlockSpec((NC, _BLK, PADW), lambda i: (0, i, 0)),
            pl.BlockSpec((_BLK, PADW), lambda i: (i, 0)),
        ],
        out_specs=pl.BlockSpec((_BLK, ow), lambda i: (i, 0)),
        out_shape=jax.ShapeDtypeStruct((N_NODES, ow), jnp.float32),
    )(msg_part, deg_part, local_logits)


def kernel(x, edge_index, W1, W2):
    zpad = jnp.zeros((N_NODES, PADW), jnp.float32)

    rows = edge_index[0]
    cols = edge_index[1]
    local_logits = _tc_mlp(x, W1, W2)
    msg1 = _sc_propagate(local_logits, rows, cols, zpad)
    logits1 = _tc_combine(msg1, msg1, local_logits, last=False)
    msg2 = _sc_propagate(logits1, rows, cols, zpad)
    return _tc_combine(msg2, msg1, local_logits, last=True)


# speedup vs baseline: 9.8735x; 1.1296x over previous
import functools

import jax
import jax.numpy as jnp
from jax import lax
from jax.experimental import pallas as pl
from jax.experimental.pallas import tpu as pltpu
from jax.experimental.pallas import tpu_sc as plsc

N_NODES = 10000
N_EDGES = 320000
D_FEAT = 128
D_HIDDEN = 128
N_CLASSES = 64
ALPHA = 0.25
PADW = 128

NC = 2
NS = 16
NW = NC * NS
EPW = N_EDGES // NW
CHUNK = 80
NITER = EPW // CHUNK
ZSLAB = 624
ZTAIL = N_NODES - NS * ZSLAB

_mesh = plsc.VectorSubcoreMesh(core_axis_name="c", subcore_axis_name="s")


def _rowwise_copy(src, dst, sid):
    base = pl.multiple_of(sid * ZSLAB, 8)
    pltpu.sync_copy(src.at[pl.ds(base, ZSLAB)], dst.at[pl.ds(base, ZSLAB)])

    @pl.when(sid == 0)
    def _():
        pltpu.sync_copy(src.at[pl.ds(NS * ZSLAB, ZTAIL)],
                        dst.at[pl.ds(NS * ZSLAB, ZTAIL)])


NBUF = 3
_NMAIN = (NITER - 5) // NBUF


@functools.partial(
    pl.kernel,
    mesh=_mesh,
    out_type=jax.ShapeDtypeStruct((NC, N_NODES, PADW), jnp.float32),
    scratch_types=(
        [pltpu.VMEM((CHUNK,), jnp.int32)] * (2 * NBUF)
        + [pltpu.VMEM((CHUNK, PADW), jnp.float32)] * NBUF
        + [pltpu.VMEM_SHARED((N_NODES, PADW), jnp.float32)]
        + [pltpu.SemaphoreType.DMA] * (3 * NBUF)
    ),
)
def _sc_propagate(logits_hbm, rows_hbm, cols_hbm, zeros_hbm, out_hbm,
                  *scr):
    cbuf = scr[0:NBUF]
    rbuf = scr[NBUF:2 * NBUF]
    gbuf = scr[2 * NBUF:3 * NBUF]
    acc_sh = scr[3 * NBUF]
    semc = scr[3 * NBUF + 1:4 * NBUF + 1]
    semr = scr[4 * NBUF + 1:5 * NBUF + 1]
    semg = scr[5 * NBUF + 1:6 * NBUF + 1]
    cid = lax.axis_index("c")
    sid = lax.axis_index("s")
    wid = cid * NS + sid

    def ebase(k):
        return pl.multiple_of(wid * EPW + k * CHUNK, 8)

    def cols_copy(k, b):
        return pltpu.make_async_copy(
            cols_hbm.at[pl.ds(ebase(k), CHUNK)], cbuf[b], semc[b])

    def rows_copy(k, b):
        return pltpu.make_async_copy(
            rows_hbm.at[pl.ds(ebase(k), CHUNK)], rbuf[b], semr[b])

    def gath_copy(b):
        return pltpu.make_async_copy(
            logits_hbm.at[cbuf[b]], gbuf[b], semg[b])

    def scat(b):
        pltpu.sync_copy(gbuf[b], acc_sh.at[rbuf[b]], add=True)

    for b in range(NBUF):
        cols_copy(b, b).start()
        rows_copy(b, b).start()
    _rowwise_copy(zeros_hbm, acc_sh, sid)
    plsc.subcore_barrier()
    for b in range(2):
        cols_copy(b, b).wait()
        gath_copy(b).start()

    def body(k, b, nxt_gather, nxt_idx):
        gath_copy(b).wait()
        if nxt_gather:
            b2 = (b + 2) % NBUF
            cols_copy(k + 2, b2).wait()
            gath_copy(b2).start()
        rows_copy(k, b).wait()
        scat(b)
        if nxt_idx:
            cols_copy(k + NBUF, b).start()
            rows_copy(k + NBUF, b).start()

    @pl.loop(0, _NMAIN)
    def _(gi):
        for b in range(NBUF):
            body(NBUF * gi + b, b, True, True)

    for k in range(NBUF * _NMAIN, NITER):
        body(k, k % NBUF, k + 2 < NITER, k + NBUF < NITER)

    plsc.subcore_barrier()
    _rowwise_copy(acc_sh, out_hbm.at[cid], sid)


_BLK = 1000
_GRID = N_NODES // _BLK


def _mlp_body(x_ref, w1_ref, w2_ref, o_ref):
    h = jnp.dot(x_ref[...], w1_ref[...],
                preferred_element_type=jnp.float32,
                precision=lax.Precision.HIGHEST)
    h = jnp.maximum(h, 0.0)
    logits = jnp.dot(h, w2_ref[...],
                     preferred_element_type=jnp.float32,
                     precision=lax.Precision.HIGHEST)
    lane = lax.broadcasted_iota(jnp.int32, (_BLK, N_CLASSES), 1)
    pad = jnp.where(lane == 0, 1.0, 0.0)
    o_ref[...] = jnp.concatenate([logits, pad], axis=1)


def _tc_mlp(x, W1, W2):
    return pl.pallas_call(
        _mlp_body,
        grid=(_GRID,),
        in_specs=[
            pl.BlockSpec((_BLK, D_FEAT), lambda i: (i, 0)),
            pl.BlockSpec((D_FEAT, D_HIDDEN), lambda i: (0, 0)),
            pl.BlockSpec((D_HIDDEN, N_CLASSES), lambda i: (0, 0)),
        ],
        out_specs=pl.BlockSpec((_BLK, PADW), lambda i: (i, 0)),
        out_shape=jax.ShapeDtypeStruct((N_NODES, PADW), jnp.float32),
    )(x, W1, W2)


def _combine_body(last, mp_ref, dp_ref, ll_ref, o_ref):
    msg = mp_ref[0, :, :N_CLASSES] + mp_ref[1, :, :N_CLASSES]
    deg = dp_ref[0, :, N_CLASSES:N_CLASSES + 1] + dp_ref[1, :, N_CLASSES:N_CLASSES + 1]
    coef = (1.0 - ALPHA) / jnp.maximum(deg, 1e-12)
    logits = coef * msg + ALPHA * ll_ref[:, :N_CLASSES]
    if last:
        mx = jnp.max(logits, axis=1, keepdims=True)
        lse = jnp.log(jnp.sum(jnp.exp(logits - mx), axis=1, keepdims=True)) + mx
        o_ref[...] = logits - lse
    else:
        lane = lax.broadcasted_iota(jnp.int32, (_BLK, N_CLASSES), 1)
        pad = jnp.where(lane == 0, 1.0, 0.0)
        o_ref[...] = jnp.concatenate([logits, pad], axis=1)


def _tc_combine(msg_part, deg_part, local_logits, last):
    ow = N_CLASSES if last else PADW
    return pl.pallas_call(
        functools.partial(_combine_body, last),
        grid=(_GRID,),
        in_specs=[
            pl.BlockSpec((NC, _BLK, PADW), lambda i: (0, i, 0)),
            pl.BlockSpec((NC, _BLK, PADW), lambda i: (0, i, 0)),
            pl.BlockSpec((_BLK, PADW), lambda i: (i, 0)),
        ],
        out_specs=pl.BlockSpec((_BLK, ow), lambda i: (i, 0)),
        out_shape=jax.ShapeDtypeStruct((N_NODES, ow), jnp.float32),
    )(msg_part, deg_part, local_logits)


def kernel(x, edge_index, W1, W2):
    zpad = jnp.zeros((N_NODES, PADW), jnp.float32)

    rows = edge_index[0]
    cols = edge_index[1]
    local_logits = _tc_mlp(x, W1, W2)
    msg1 = _sc_propagate(local_logits, rows, cols, zpad)
    logits1 = _tc_combine(msg1, msg1, local_logits, last=False)
    msg2 = _sc_propagate(logits1, rows, cols, zpad)
    return _tc_combine(msg2, msg1, local_logits, last=True)
